# baseline (speedup 1.0000x reference)
import functools

import jax
import jax.numpy as jnp
from jax import lax
from jax.experimental import pallas as pl
from jax.experimental.pallas import tpu as pltpu
from jax.experimental.pallas import tpu_sc as plsc

_K = 4335206
_ROWS = 8
_ROWLEN = 96 * 224 * 224


def _keys_from_f32(x):
    u = jax.lax.bitcast_convert_type(x, jnp.uint32)
    return jnp.where((u >> 31) == 1, ~u, u ^ jnp.uint32(0x80000000))


def _f32_from_key(k):
    u = jnp.where((k >> 31) == 1, k ^ jnp.uint32(0x80000000), ~k)
    return jax.lax.bitcast_convert_type(u, jnp.float32)


def _thresh_body(nb, kth, x_ref, out_ref, prefix_ref, krem_ref, acc_ref):
    r = pl.program_id(0)
    p = pl.program_id(1)
    b = pl.program_id(2)

    @pl.when((p == 0) & (b == 0))
    def _init():
        prefix_ref[r] = jnp.uint32(0)
        krem_ref[r] = jnp.int32(kth)
        acc_ref[r] = jnp.int32(0)

    shift = (jnp.int32(31) - p).astype(jnp.uint32)
    key = _keys_from_f32(x_ref[0])
    prefix = prefix_ref[r]
    mask_hi = ~((jnp.uint32(2) << shift) - jnp.uint32(1))
    inr = (key & mask_hi) == prefix
    bit0 = ((key >> shift) & jnp.uint32(1)) == jnp.uint32(0)
    c0 = jnp.sum((inr & bit0).astype(jnp.int32))
    acc_ref[r] = acc_ref[r] + c0

    @pl.when(b == nb - 1)
    def _decide():
        c0_tot = acc_ref[r]
        krem = krem_ref[r]
        take_one = c0_tot < krem
        new_prefix = jnp.where(take_one, prefix | (jnp.uint32(1) << shift), prefix)
        krem_ref[r] = jnp.where(take_one, krem - c0_tot, krem)
        prefix_ref[r] = new_prefix
        acc_ref[r] = jnp.int32(0)

        @pl.when(p == 31)
        def _emit():
            out_ref[0, 0, :] = jnp.full((128,), _f32_from_key(new_prefix), jnp.float32)


def _mask_body(x_ref, thr_ref, o_ref):
    t = thr_ref[0, 0, 0]
    x = x_ref[...]
    o_ref[...] = x * (x >= t).astype(jnp.float32)


def _kthvalue_mask(x, rows, rowlen, kth, sub, lanes, nb):
    xv = x.reshape(rows, sub, lanes)
    sub_blk = sub // nb

    thr = pl.pallas_call(
        lambda *a: _thresh_body(nb, kth, *a),
        grid=(rows, 32, nb),
        in_specs=[pl.BlockSpec((1, sub_blk, lanes), lambda r, p, b: (r, b, 0))],
        out_specs=pl.BlockSpec((1, 1, 128), lambda r, p, b: (r, 0, 0)),
        out_shape=jax.ShapeDtypeStruct((rows, 1, 128), jnp.float32),
        scratch_shapes=[
            pltpu.SMEM((rows,), jnp.uint32),
            pltpu.SMEM((rows,), jnp.int32),
            pltpu.SMEM((rows,), jnp.int32),
        ],
    )(xv)

    out = _apply_mask(xv, thr[:, :, :16], rows, sub, lanes, nb, sub_blk)
    return out.reshape(rows, rowlen)


def _apply_mask(xv, thr16, rows, sub, lanes, nb, sub_blk):
    return pl.pallas_call(
        _mask_body,
        grid=(rows, nb),
        in_specs=[
            pl.BlockSpec((1, sub_blk, lanes), lambda r, b: (r, b, 0)),
            pl.BlockSpec((1, 1, 16), lambda r, b: (r, 0, 0)),
        ],
        out_specs=pl.BlockSpec((1, sub_blk, lanes), lambda r, b: (r, b, 0)),
        out_shape=jax.ShapeDtypeStruct((rows, sub, lanes), jnp.float32),
    )(xv, thr16)


def _make_sc_thresh(rowlen, chunk, kth, interpret=False):
    qlen = rowlen // 4
    nchunk = qlen // chunk
    assert nchunk * chunk == qlen
    nb = 4096
    cap = 16384

    mesh = plsc.VectorSubcoreMesh(core_axis_name="c", subcore_axis_name="s",
                                  num_cores=2, num_subcores=16)

    @functools.partial(
        pl.kernel,
        out_type=jax.ShapeDtypeStruct((8, 16), jnp.uint32),
        mesh=mesh,
        scratch_types=[
            pltpu.VMEM((16 * nb,), jnp.int32),
            pltpu.VMEM((chunk,), jnp.uint32),
            pltpu.VMEM((nb,), jnp.int32),
            pltpu.VMEM((4, nb), jnp.int32),
            pltpu.VMEM((16,), jnp.uint32),
            pltpu.VMEM((cap + 16,), jnp.uint32),
            pltpu.VMEM_SHARED((16, nb), jnp.int32),
        ],
        compiler_params=pltpu.CompilerParams(needs_layout_passes=False),
        interpret=interpret,
    )
    def sc_thresh(x_hbm, out_hbm, hist, buf, red, grp, obuf, cbuf, shist):
        c = lax.axis_index("c")
        s = lax.axis_index("s")
        lrow = s // 4
        row = c * 4 + lrow
        q = s % 4
        base = row * rowlen + q * qlen
        iota16 = lax.iota(jnp.int32, 16)
        ones = jnp.ones((16,), jnp.int32)
        laneoff = iota16 * nb

        def monotone(u):
            return jnp.where((u >> jnp.uint32(31)) == jnp.uint32(1),
                             ~u, u ^ jnp.uint32(0x80000000))

        def zero_hist():
            def zbody(i, _):
                hist[pl.ds(i * 16, 16)] = jnp.zeros((16,), jnp.int32)
                return 0
            lax.fori_loop(0, nb, zbody, 0, unroll=8)

        def merge_and_scan(krem_s):
            def rbody(vi, _):
                acc = hist[pl.ds(vi * 16, 16)]
                for l in range(1, 16):
                    acc = acc + hist[pl.ds(l * nb + vi * 16, 16)]
                red[pl.ds(vi * 16, 16)] = acc
                return 0
            lax.fori_loop(0, nb // 16, rbody, 0)

            pltpu.sync_copy(red, shist.at[s])
            plsc.subcore_barrier()
            pltpu.sync_copy(shist.at[pl.ds(lrow * 4, 4)], grp)

            def mbody(vi, _):
                acc = (grp[0, pl.ds(vi * 16, 16)] +
                       grp[1, pl.ds(vi * 16, 16)] +
                       grp[2, pl.ds(vi * 16, 16)] +
                       grp[3, pl.ds(vi * 16, 16)])
                red[pl.ds(vi * 16, 16)] = acc
                return 0
            lax.fori_loop(0, nb // 16, mbody, 0)
            plsc.subcore_barrier()

            def sbody(vi, carry):
                done, binv, below, tot = carry
                vec = red[pl.ds(vi * 16, 16)]
                cum = plsc.cumsum(vec)
                g = tot + cum
                m = g >= krem_s
                many = jnp.sum(m.astype(jnp.int32))
                ffs = jnp.min(jnp.where(m, iota16, jnp.int32(16)))
                below_here = tot + jnp.sum(jnp.where(iota16 < ffs, vec, 0))
                found_now = (many > 0) & (done == 0)
                binv = jnp.where(found_now, vi * 16 + ffs, binv)
                below = jnp.where(found_now, below_here, below)
                done = jnp.where(many > 0, jnp.int32(1), done)
                tot = tot + jnp.sum(vec)
                return (done, binv, below, tot)

            init = (jnp.int32(0), jnp.int32(0), jnp.int32(0), jnp.int32(0))
            _, binv, below, _ = lax.fori_loop(0, nb // 16, sbody, init)
            return binv, below

        zero_hist()

        def cbody1(ci, _):
            pltpu.sync_copy(x_hbm.at[pl.ds(base + ci * chunk, chunk)], buf)

            def vbody(j, _):
                key = monotone(buf[pl.ds(j * 16, 16)])
                idx = (key >> jnp.uint32(20)).astype(jnp.int32)
                plsc.addupdate_scatter(hist, [laneoff + idx], ones)
                return 0
            lax.fori_loop(0, chunk // 16, vbody, 0, unroll=8)
            return 0
        lax.fori_loop(0, nchunk, cbody1, 0)

        b1, below1 = merge_and_scan(jnp.int32(kth))
        krem1 = jnp.int32(kth) - below1

        zero_hist()
        b1v = jnp.full((16,), b1, jnp.int32)

        def cbody2(ci, off):
            pltpu.sync_copy(x_hbm.at[pl.ds(base + ci * chunk, chunk)], buf)

            def vbody(j, off):
                key = monotone(buf[pl.ds(j * 16, 16)])
                m = (key >> jnp.uint32(20)).astype(jnp.int32) == b1v
                idx = ((key >> jnp.uint32(8)) & jnp.uint32(0xFFF)).astype(jnp.int32)
                plsc.addupdate_scatter(hist, [laneoff + idx], ones, mask=m)
                plsc.store_compressed(cbuf.at[pl.ds(jnp.minimum(off, cap), 16)],
                                      key, mask=m)
                return off + jnp.sum(m.astype(jnp.int32))
            return lax.fori_loop(0, chunk // 16, vbody, off, unroll=8)
        off = lax.fori_loop(0, nchunk, cbody2, jnp.int32(0))

        b2, below2 = merge_and_scan(krem1)
        krem2 = krem1 - below2

        zero_hist()
        b2v = jnp.full((16,), b2, jnp.int32)

        @pl.when(off <= cap)
        def _fast():
            def vbody(j, _):
                key = cbuf[pl.ds(j * 16, 16)]
                valid = (j * 16 + iota16) < off
                m = (((key >> jnp.uint32(8)) & jnp.uint32(0xFFF))
                     .astype(jnp.int32) == b2v) & valid
                idx = (key & jnp.uint32(0xFF)).astype(jnp.int32)
                plsc.addupdate_scatter(hist, [laneoff + idx], ones, mask=m)
                return 0
            lax.fori_loop(0, cap // 16, vbody, 0, unroll=8)

        @pl.when(off > cap)
        def _slow():
            p24 = jnp.full((16,), (b1 << jnp.int32(12)) | b2, jnp.int32)

            def cbody3(ci, _):
                pltpu.sync_copy(x_hbm.at[pl.ds(base + ci * chunk, chunk)], buf)

                def vbody(j, _):
                    key = monotone(buf[pl.ds(j * 16, 16)])
                    m = (key >> jnp.uint32(8)).astype(jnp.int32) == p24
                    idx = (key & jnp.uint32(0xFF)).astype(jnp.int32)
                    plsc.addupdate_scatter(hist, [laneoff + idx], ones, mask=m)
                    return 0
                lax.fori_loop(0, chunk // 16, vbody, 0, unroll=8)
                return 0
            lax.fori_loop(0, nchunk, cbody3, 0)

        b3, below3 = merge_and_scan(krem2)
        del below3

        prefix_u = ((b1.astype(jnp.uint32) << jnp.uint32(20))
                    | (b2.astype(jnp.uint32) << jnp.uint32(8))
                    | b3.astype(jnp.uint32))

        @pl.when(q == 0)
        def _emit():
            obuf[...] = jnp.full((16,), prefix_u, jnp.uint32)
            pltpu.sync_copy(obuf, out_hbm.at[row])

    return sc_thresh


def kernel(inputs):
    b = inputs.shape[0]
    flat = inputs.reshape(b, _ROWLEN)
    bits = jax.lax.bitcast_convert_type(flat, jnp.uint32)
    key16 = _make_sc_thresh(_ROWLEN, 8192, _K)(bits.reshape(-1))
    thr16 = _f32_from_key(key16)
    xv = flat.reshape(b, 768, 6272)
    out = _apply_mask(xv, thr16.reshape(b, 1, 16), b, 768, 6272, 8, 96)
    return out.reshape(inputs.shape)


# speedup vs baseline: 1.4068x; 1.4068x over previous
import functools

import jax
import jax.numpy as jnp
from jax import lax
from jax.experimental import pallas as pl
from jax.experimental.pallas import tpu as pltpu
from jax.experimental.pallas import tpu_sc as plsc

_K = 4335206
_ROWS = 8
_ROWLEN = 96 * 224 * 224


def _f32_from_key(k):
    u = jnp.where((k >> 31) == 1, k ^ jnp.uint32(0x80000000), ~k)
    return jax.lax.bitcast_convert_type(u, jnp.float32)


def _mask_body(x_ref, thr_ref, o_ref):
    t = thr_ref[0, 0, 0]
    x = x_ref[...]
    o_ref[...] = x * (x >= t).astype(jnp.float32)


def _apply_mask(xv, thr16, rows, sub, lanes, nb, sub_blk):
    return pl.pallas_call(
        _mask_body,
        grid=(rows, nb),
        in_specs=[
            pl.BlockSpec((1, sub_blk, lanes), lambda r, b: (r, b, 0)),
            pl.BlockSpec((1, 1, 16), lambda r, b: (r, 0, 0)),
        ],
        out_specs=pl.BlockSpec((1, sub_blk, lanes), lambda r, b: (r, b, 0)),
        out_shape=jax.ShapeDtypeStruct((rows, sub, lanes), jnp.float32),
    )(xv, thr16)


def _make_sc_thresh(rowlen, chunk, kth, interpret=False):
    qlen = rowlen // 4
    nchunk = qlen // chunk
    assert nchunk * chunk == qlen
    nb = 4096
    cap = 32768

    mesh = plsc.VectorSubcoreMesh(core_axis_name="c", subcore_axis_name="s",
                                  num_cores=2, num_subcores=16)
    signbit = jnp.int32(-2147483648)

    @functools.partial(
        pl.kernel,
        out_type=jax.ShapeDtypeStruct((8, 16), jnp.int32),
        mesh=mesh,
        scratch_types=[
            pltpu.VMEM((nb,), jnp.int32),
            pltpu.VMEM((chunk,), jnp.int32),
            pltpu.VMEM((nb,), jnp.int32),
            pltpu.VMEM((4, nb), jnp.int32),
            pltpu.VMEM((16,), jnp.int32),
            pltpu.VMEM((cap + 16,), jnp.int32),
            pltpu.VMEM_SHARED((16, nb), jnp.int32),
        ],
        compiler_params=pltpu.CompilerParams(needs_layout_passes=False),
        interpret=interpret,
    )
    def sc_thresh(x_hbm, out_hbm, hist, buf, red, grp, obuf, cbuf, shist):
        c = lax.axis_index("c")
        s = lax.axis_index("s")
        lrow = s // 4
        row = c * 4 + lrow
        q = s % 4
        base = row * rowlen + q * qlen
        iota16 = lax.iota(jnp.int32, 16)
        ones = jnp.ones((16,), jnp.int32)

        def monotone(u):
            return u ^ ((u >> jnp.int32(31)) | signbit)

        def zero_hist():
            def zbody(i, _):
                hist[pl.ds(i * 16, 16)] = jnp.zeros((16,), jnp.int32)
                return 0
            lax.fori_loop(0, nb // 16, zbody, 0, unroll=8)

        def merge_and_scan(krem_s):
            pltpu.sync_copy(hist, shist.at[s])
            plsc.subcore_barrier()
            pltpu.sync_copy(shist.at[pl.ds(lrow * 4, 4)], grp)

            def mbody(vi, _):
                acc = (grp[0, pl.ds(vi * 16, 16)] +
                       grp[1, pl.ds(vi * 16, 16)] +
                       grp[2, pl.ds(vi * 16, 16)] +
                       grp[3, pl.ds(vi * 16, 16)])
                red[pl.ds(vi * 16, 16)] = acc
                return 0
            lax.fori_loop(0, nb // 16, mbody, 0)
            plsc.subcore_barrier()

            def sbody(vi, carry):
                done, binv, below, tot = carry
                vec = red[pl.ds(vi * 16, 16)]
                cum = plsc.cumsum(vec)
                g = tot + cum
                m = g >= krem_s
                many = jnp.sum(m.astype(jnp.int32))
                ffs = jnp.min(jnp.where(m, iota16, jnp.int32(16)))
                below_here = tot + jnp.sum(jnp.where(iota16 < ffs, vec, 0))
                found_now = (many > 0) & (done == 0)
                binv = jnp.where(found_now, vi * 16 + ffs, binv)
                below = jnp.where(found_now, below_here, below)
                done = jnp.where(many > 0, jnp.int32(1), done)
                tot = tot + jnp.sum(vec)
                return (done, binv, below, tot)

            init = (jnp.int32(0), jnp.int32(0), jnp.int32(0), jnp.int32(0))
            _, binv, below, _ = lax.fori_loop(0, nb // 16, sbody, init)
            return binv, below

        zero_hist()

        def cbody1(ci, _):
            pltpu.sync_copy(x_hbm.at[pl.ds(base + ci * chunk, chunk)], buf)

            def vbody(j, _):
                key = monotone(buf[pl.ds(j * 16, 16)])
                idx = (key >> jnp.int32(20)) & jnp.int32(0xFFF)
                plsc.addupdate_scatter(hist, [idx], ones)
                return 0
            lax.fori_loop(0, chunk // 16, vbody, 0, unroll=8)
            return 0
        lax.fori_loop(0, nchunk, cbody1, 0)

        b1, below1 = merge_and_scan(jnp.int32(kth))
        krem1 = jnp.int32(kth) - below1

        zero_hist()
        b1v = jnp.full((16,), b1, jnp.int32)

        def cbody2(ci, off):
            pltpu.sync_copy(x_hbm.at[pl.ds(base + ci * chunk, chunk)], buf)

            def vbody(j, off):
                key = monotone(buf[pl.ds(j * 16, 16)])
                m = ((key >> jnp.int32(20)) & jnp.int32(0xFFF)) == b1v
                idx = (key >> jnp.int32(8)) & jnp.int32(0xFFF)
                plsc.addupdate_scatter(hist, [idx], ones, mask=m)
                plsc.store_compressed(cbuf.at[pl.ds(jnp.minimum(off, cap), 16)],
                                      key, mask=m)
                return off + jnp.sum(m.astype(jnp.int32))
            return lax.fori_loop(0, chunk // 16, vbody, off, unroll=8)
        off = lax.fori_loop(0, nchunk, cbody2, jnp.int32(0))

        b2, below2 = merge_and_scan(krem1)
        krem2 = krem1 - below2

        zero_hist()
        b2v = jnp.full((16,), b2, jnp.int32)

        @pl.when(off <= cap)
        def _fast():
            def vbody(j, _):
                key = cbuf[pl.ds(j * 16, 16)]
                valid = (j * 16 + iota16) < off
                m = (((key >> jnp.int32(8)) & jnp.int32(0xFFF)) == b2v) & valid
                idx = key & jnp.int32(0xFF)
                plsc.addupdate_scatter(hist, [idx], ones, mask=m)
                return 0
            lax.fori_loop(0, cap // 16, vbody, 0, unroll=8)

        @pl.when(off > cap)
        def _slow():
            p24 = jnp.full((16,), (b1 << jnp.int32(12)) | b2, jnp.int32)

            def cbody3(ci, _):
                pltpu.sync_copy(x_hbm.at[pl.ds(base + ci * chunk, chunk)], buf)

                def vbody(j, _):
                    key = monotone(buf[pl.ds(j * 16, 16)])
                    m = ((key >> jnp.int32(8)) & jnp.int32(0xFFFFFF)) == p24
                    idx = key & jnp.int32(0xFF)
                    plsc.addupdate_scatter(hist, [idx], ones, mask=m)
                    return 0
                lax.fori_loop(0, chunk // 16, vbody, 0, unroll=8)
                return 0
            lax.fori_loop(0, nchunk, cbody3, 0)

        b3, below3 = merge_and_scan(krem2)
        del below3

        prefix = (b1 << jnp.int32(20)) | (b2 << jnp.int32(8)) | b3

        @pl.when(q == 0)
        def _emit():
            obuf[...] = jnp.full((16,), prefix, jnp.int32)
            pltpu.sync_copy(obuf, out_hbm.at[row])

    return sc_thresh


def kernel(inputs):
    b = inputs.shape[0]
    flat = inputs.reshape(b, _ROWLEN)
    bits = jax.lax.bitcast_convert_type(flat, jnp.int32)
    key16 = _make_sc_thresh(_ROWLEN, 8192, _K)(bits.reshape(-1))
    thr16 = _f32_from_key(jax.lax.bitcast_convert_type(key16, jnp.uint32))
    xv = flat.reshape(b, 768, 6272)
    out = _apply_mask(xv, thr16.reshape(b, 1, 16), b, 768, 6272, 8, 96)
    return out.reshape(inputs.shape)
